# baseline (speedup 1.0000x reference)
import functools

import jax
import jax.numpy as jnp
from jax import lax
from jax.experimental import pallas as pl
from jax.experimental.pallas import tpu as pltpu
from jax.experimental.pallas import tpu_sc as plsc

B = 2
N = 8192
K = 16
EPS = 1e-6
TD = 1024
TL = 256
TN = 256
ND = N // TD
NL = N // TL
NK = N // TN
TC = 48
BNK = B * N * K


def _dot(a, b):
    return lax.dot_general(a, b, (((1,), (1,)), ((), ())),
                           preferred_element_type=jnp.float32)


def _leaky(x, slope):
    return jnp.where(x >= 0, x, slope * x)


def _gn_scale(ssum, ssq, count, c, gamma, beta):
    s = jnp.sum(ssum, axis=0, keepdims=True)
    q = jnp.sum(ssq, axis=0, keepdims=True)
    gper = c // 16
    gi = lax.broadcasted_iota(jnp.int32, (c, c), 0) // gper
    gj = lax.broadcasted_iota(jnp.int32, (c, c), 1) // gper
    gmat = (gi == gj).astype(jnp.float32)
    mu = _dot(s, gmat) / count
    msq = _dot(q, gmat) / count
    inv = lax.rsqrt(msq - mu * mu + EPS)
    scale = inv * gamma
    shift = beta - mu * scale
    return scale, shift


def _acc(ref, val, is_first):
    @pl.when(is_first)
    def _():
        ref[...] = val[None]

    @pl.when(jnp.logical_not(is_first))
    def _():
        ref[...] += val[None]


def _k0_body(f_ref, c_ref, w1_ref, b1_ref, scw_ref, scb_ref,
             tab_ref, ssum_ref, ssq_ref):
    t = pl.program_id(1)
    f = f_ref[0]
    c = c_ref[0]
    x1 = _leaky(_dot(f, w1_ref[...]) + b1_ref[...], 0.2)
    tab_ref[...] = jnp.concatenate(
        [x1, c, jnp.zeros((TD, TC - 35), jnp.float32)], axis=1)
    pre = _dot(f, scw_ref[...]) + scb_ref[...]
    ps = jnp.sum(pre.reshape(TD // 8, 8, 128), axis=0)
    pq = jnp.sum((pre * pre).reshape(TD // 8, 8, 128), axis=0)
    _acc(ssum_ref, ps, t == 0)
    _acc(ssq_ref, pq, t == 0)


def _k0(feats, coords, w1, b1, scw, scb):
    return pl.pallas_call(
        _k0_body,
        grid=(B, ND),
        in_specs=[
            pl.BlockSpec((1, TD, 32), lambda b, t: (b, t, 0)),
            pl.BlockSpec((1, TD, 3), lambda b, t: (b, t, 0)),
            pl.BlockSpec((32, 32), lambda b, t: (0, 0)),
            pl.BlockSpec((1, 32), lambda b, t: (0, 0)),
            pl.BlockSpec((128, 32), lambda b, t: (0, 0)),
            pl.BlockSpec((1, 128), lambda b, t: (0, 0)),
        ],
        out_specs=[
            pl.BlockSpec((TD, TC), lambda b, t: (b * ND + t, 0)),
            pl.BlockSpec((1, 8, 128), lambda b, t: (b, 0, 0)),
            pl.BlockSpec((1, 8, 128), lambda b, t: (b, 0, 0)),
        ],
        out_shape=[
            jax.ShapeDtypeStruct((B * N, TC), jnp.float32),
            jax.ShapeDtypeStruct((B, 8, 128), jnp.float32),
            jax.ShapeDtypeStruct((B, 8, 128), jnp.float32),
        ],
    )(feats, coords, w1, b1, scw, scb)


def _knn_body(r_ref, a_ref, idx_ref):
    b = pl.program_id(0)
    r = r_ref[0]
    allc = a_ref[0]
    sr = jnp.sum(r * r, axis=1)
    sa = jnp.sum(allc * allc, axis=1)
    cross = _dot(r, allc)
    d2 = sr[:, None] + sa[None, :] - 2.0 * cross
    colid = lax.broadcasted_iota(jnp.int32, (TN, N), 1)
    for k in range(K):
        a = jnp.argmin(d2, axis=1).astype(jnp.int32)
        idx_ref[0, k, :] = a + b * N
        d2 = jnp.where(colid == a[:, None], jnp.float32(jnp.inf), d2)


def _knn(coords):
    return pl.pallas_call(
        _knn_body,
        grid=(B, NK),
        in_specs=[
            pl.BlockSpec((1, TN, 3), lambda b, t: (b, t, 0)),
            pl.BlockSpec((1, N, 3), lambda b, t: (b, 0, 0)),
        ],
        out_specs=pl.BlockSpec((1, K, TN), lambda b, t: (b, 0, t)),
        out_shape=jax.ShapeDtypeStruct((B, K, N), jnp.int32),
    )(coords, coords)


_SC_NW = 32
_SC_RW = BNK // _SC_NW
_SC_SUP = 1024
_SC_NSUP = _SC_RW // _SC_SUP


def _sc_gather_body(tab_ref, idx_ref, out_ref, idx_v, rows_v, sem):
    wid = lax.axis_index("s") * 2 + lax.axis_index("c")
    for s in range(_SC_NSUP):
        base = pl.multiple_of(wid * _SC_RW + s * _SC_SUP, _SC_SUP)
        irow = pl.multiple_of(base // 128, _SC_SUP // 128)
        pltpu.sync_copy(idx_ref.at[pl.ds(irow, _SC_SUP // 128)], idx_v)
        copies = []
        for j in range(_SC_SUP // 128):
            copies.append(pltpu.async_copy(
                tab_ref.at[idx_v.at[j]],
                rows_v.at[pl.ds(j * 128, 128)], sem))
        for cp in copies:
            cp.wait()
        pltpu.sync_copy(rows_v, out_ref.at[pl.ds(base, _SC_SUP)])


@functools.cache
def _sc_gather_kernel():
    return functools.partial(
        pl.kernel,
        mesh=plsc.VectorSubcoreMesh(core_axis_name="c", subcore_axis_name="s"),
        compiler_params=pltpu.CompilerParams(use_tc_tiling_on_sc=False),
        out_type=jax.ShapeDtypeStruct((BNK, TC), jnp.float32),
        scratch_types=[
            pltpu.VMEM((_SC_SUP // 128, 128), jnp.int32),
            pltpu.VMEM((_SC_SUP, TC), jnp.float32),
            pltpu.SemaphoreType.DMA,
        ],
    )(_sc_gather_body)


def _run_gather(table, idx2d):
    return _sc_gather_kernel()(table, idx2d)


def _lse_pre(c, g, w, bias):
    wn = w[:, 0:3] + w[:, 6:9]
    wnb = w[:, 3:6] - w[:, 6:9]
    wd3 = jnp.broadcast_to(w[:, 9:10], (32, 3))
    a_n = _dot(c, wn)
    c_nb = g[:, 32:35]
    a_nb = _dot(c_nb, wnb)
    c_self = jnp.broadcast_to(c[:, None, :], (TL, K, 3)).reshape(TL * K, 3)
    diff = c_self - c_nb
    dterm = _dot(diff * diff, wd3)
    a_nr = jnp.broadcast_to(a_n[:, None, :], (TL, K, 32)).reshape(TL * K, 32)
    return a_nr + a_nb + dterm + bias


def _lseA_body(c_ref, g_ref, w_ref, b_ref,
               ssum_ref, ssq_ref, fmean_ref, pre_ref):
    t = pl.program_id(1)
    g = g_ref[...]
    pre = _lse_pre(c_ref[0], g, w_ref[...], b_ref[...])
    pre_ref[...] = pre
    fmean_ref[0] = jnp.mean(g[:, 0:32].reshape(TL, K, 32), axis=1)
    ps = jnp.sum(pre.reshape(TL * K // 8, 8, 32), axis=0)
    pq = jnp.sum((pre * pre).reshape(TL * K // 8, 8, 32), axis=0)
    _acc(ssum_ref, ps, t == 0)
    _acc(ssq_ref, pq, t == 0)


def _lseA(coords, gath, w, bias):
    return pl.pallas_call(
        _lseA_body,
        grid=(B, NL),
        in_specs=[
            pl.BlockSpec((1, TL, 3), lambda b, t: (b, t, 0)),
            pl.BlockSpec((TL * K, TC), lambda b, t: (b * NL + t, 0)),
            pl.BlockSpec((32, 10), lambda b, t: (0, 0)),
            pl.BlockSpec((1, 32), lambda b, t: (0, 0)),
        ],
        out_specs=[
            pl.BlockSpec((1, 8, 32), lambda b, t: (b, 0, 0)),
            pl.BlockSpec((1, 8, 32), lambda b, t: (b, 0, 0)),
            pl.BlockSpec((1, TL, 32), lambda b, t: (b, t, 0)),
            pl.BlockSpec((TL * K, 32), lambda b, t: (b * NL + t, 0)),
        ],
        out_shape=[
            jax.ShapeDtypeStruct((B, 8, 32), jnp.float32),
            jax.ShapeDtypeStruct((B, 8, 32), jnp.float32),
            jax.ShapeDtypeStruct((B, N, 32), jnp.float32),
            jax.ShapeDtypeStruct((BNK, 32), jnp.float32),
        ],
    )(coords, gath, w, bias)


def _lseB_body(pout, pre_ref, gam_ref, bet_ref,
               ssum_ref, ssq_ref, fmean_ref, pw_ref, pb_ref,
               prep_ref, psum_ref, psq_ref):
    t = pl.program_id(1)
    scale, shift = _gn_scale(ssum_ref[0], ssq_ref[0], 2.0 * N * K, 32,
                             gam_ref[...], bet_ref[...])
    h = jax.nn.relu(pre_ref[...] * scale + shift)
    hmean = jnp.mean(h.reshape(TL, K, 32), axis=1)
    pin = jnp.concatenate([hmean, fmean_ref[0]], axis=1)
    prep = _dot(pin, pw_ref[...]) + pb_ref[...]
    prep_ref[0] = prep
    ps = jnp.sum(prep.reshape(TL // 8, 8, pout), axis=0)
    pq = jnp.sum((prep * prep).reshape(TL // 8, 8, pout), axis=0)
    _acc(psum_ref, ps, t == 0)
    _acc(psq_ref, pq, t == 0)


def _lseB(pout, pre, gam, bet, ssum, ssq, fmean, pw, pb):
    return pl.pallas_call(
        functools.partial(_lseB_body, pout),
        grid=(B, NL),
        in_specs=[
            pl.BlockSpec((TL * K, 32), lambda b, t: (b * NL + t, 0)),
            pl.BlockSpec((1, 32), lambda b, t: (0, 0)),
            pl.BlockSpec((1, 32), lambda b, t: (0, 0)),
            pl.BlockSpec((1, 8, 32), lambda b, t: (b, 0, 0)),
            pl.BlockSpec((1, 8, 32), lambda b, t: (b, 0, 0)),
            pl.BlockSpec((1, TL, 32), lambda b, t: (b, t, 0)),
            pl.BlockSpec((pout, 64), lambda b, t: (0, 0)),
            pl.BlockSpec((1, pout), lambda b, t: (0, 0)),
        ],
        out_specs=[
            pl.BlockSpec((1, TL, pout), lambda b, t: (b, t, 0)),
            pl.BlockSpec((1, 8, pout), lambda b, t: (b, 0, 0)),
            pl.BlockSpec((1, 8, pout), lambda b, t: (b, 0, 0)),
        ],
        out_shape=[
            jax.ShapeDtypeStruct((B, N, pout), jnp.float32),
            jax.ShapeDtypeStruct((B, 8, pout), jnp.float32),
            jax.ShapeDtypeStruct((B, 8, pout), jnp.float32),
        ],
    )(pre, gam, bet, ssum, ssq, fmean, pw, pb)


def _k5_body(c_ref, prep_ref, ssum_ref, ssq_ref, gam_ref, bet_ref, tab_ref):
    scale, shift = _gn_scale(ssum_ref[0], ssq_ref[0], 2.0 * N, 32,
                             gam_ref[...], bet_ref[...])
    x2 = jax.nn.relu(prep_ref[0] * scale + shift)
    tab_ref[...] = jnp.concatenate(
        [x2, c_ref[0], jnp.zeros((TD, TC - 35), jnp.float32)], axis=1)


def _k5(coords, prep, ssum, ssq, gam, bet):
    return pl.pallas_call(
        _k5_body,
        grid=(B, ND),
        in_specs=[
            pl.BlockSpec((1, TD, 3), lambda b, t: (b, t, 0)),
            pl.BlockSpec((1, TD, 32), lambda b, t: (b, t, 0)),
            pl.BlockSpec((1, 8, 32), lambda b, t: (b, 0, 0)),
            pl.BlockSpec((1, 8, 32), lambda b, t: (b, 0, 0)),
            pl.BlockSpec((1, 32), lambda b, t: (0, 0)),
            pl.BlockSpec((1, 32), lambda b, t: (0, 0)),
        ],
        out_specs=pl.BlockSpec((TD, TC), lambda b, t: (b * ND + t, 0)),
        out_shape=jax.ShapeDtypeStruct((B * N, TC), jnp.float32),
    )(coords, prep, ssum, ssq, gam, bet)


def _k9_body(prep_ref, p2s_ref, p2q_ref, g2_ref, b2_ref,
             m2w_ref, m2b_ref, f_ref, scw_ref, scb_ref,
             scs_ref, scq_ref, scg_ref, scbt_ref, out_ref):
    scale, shift = _gn_scale(p2s_ref[0], p2q_ref[0], 4.0 * N, 64,
                             g2_ref[...], b2_ref[...])
    x3 = jax.nn.relu(prep_ref[0] * scale + shift)
    main = _dot(x3, m2w_ref[...]) + m2b_ref[...]
    pre_sc = _dot(f_ref[0], scw_ref[...]) + scb_ref[...]
    scale2, shift2 = _gn_scale(scs_ref[0], scq_ref[0], 8.0 * N, 128,
                               scg_ref[...], scbt_ref[...])
    scn = pre_sc * scale2 + shift2
    out_ref[0] = _leaky(main + scn, 0.01)


def _k9(prep2, p2s, p2q, g2, b2, m2w, m2b, feats, scw, scb, scs, scq,
        scg, scbt):
    return pl.pallas_call(
        _k9_body,
        grid=(B, ND),
        in_specs=[
            pl.BlockSpec((1, TD, 64), lambda b, t: (b, t, 0)),
            pl.BlockSpec((1, 8, 64), lambda b, t: (b, 0, 0)),
            pl.BlockSpec((1, 8, 64), lambda b, t: (b, 0, 0)),
            pl.BlockSpec((1, 64), lambda b, t: (0, 0)),
            pl.BlockSpec((1, 64), lambda b, t: (0, 0)),
            pl.BlockSpec((128, 64), lambda b, t: (0, 0)),
            pl.BlockSpec((1, 128), lambda b, t: (0, 0)),
            pl.BlockSpec((1, TD, 32), lambda b, t: (b, t, 0)),
            pl.BlockSpec((128, 32), lambda b, t: (0, 0)),
            pl.BlockSpec((1, 128), lambda b, t: (0, 0)),
            pl.BlockSpec((1, 8, 128), lambda b, t: (b, 0, 0)),
            pl.BlockSpec((1, 8, 128), lambda b, t: (b, 0, 0)),
            pl.BlockSpec((1, 128), lambda b, t: (0, 0)),
            pl.BlockSpec((1, 128), lambda b, t: (0, 0)),
        ],
        out_specs=pl.BlockSpec((1, TD, 128), lambda b, t: (b, t, 0)),
        out_shape=jax.ShapeDtypeStruct((B, N, 128), jnp.float32),
    )(prep2, p2s, p2q, g2, b2, m2w, m2b, feats, scw, scb, scs, scq,
      scg, scbt)


def kernel(coords, features, mlp1_W, mlp1_b, lse1_W, lse1_b, lse1_gamma,
           lse1_beta, pool1_W, pool1_b, pool1_gamma, pool1_beta, lse2_W,
           lse2_b, lse2_gamma, lse2_beta, pool2_W, pool2_b, pool2_gamma,
           pool2_beta, mlp2_W, mlp2_b, sc_W, sc_b, sc_gamma, sc_beta):
    r1 = lambda v: v.reshape(1, -1)
    feats = jnp.transpose(features[:, :, :, 0], (0, 2, 1))

    table1, sc_sum, sc_sq = _k0(feats, coords, mlp1_W, r1(mlp1_b),
                                sc_W, r1(sc_b))
    idx_kn = _knn(coords)
    idx2d = jnp.transpose(idx_kn, (0, 2, 1)).reshape(BNK // 128, 128)

    gath1 = _run_gather(table1, idx2d)
    s1, q1, fm1, pre1 = _lseA(coords, gath1, lse1_W, r1(lse1_b))
    prep1, p1s, p1q = _lseB(32, pre1,
                            r1(lse1_gamma), r1(lse1_beta), s1, q1, fm1,
                            pool1_W, r1(pool1_b))
    table2 = _k5(coords, prep1, p1s, p1q, r1(pool1_gamma), r1(pool1_beta))

    gath2 = _run_gather(table2, idx2d)
    s2, q2, fm2, pre2 = _lseA(coords, gath2, lse2_W, r1(lse2_b))
    prep2, p2s, p2q = _lseB(64, pre2,
                            r1(lse2_gamma), r1(lse2_beta), s2, q2, fm2,
                            pool2_W, r1(pool2_b))

    out = _k9(prep2, p2s, p2q, r1(pool2_gamma), r1(pool2_beta), mlp2_W,
              r1(mlp2_b), feats, sc_W, r1(sc_b), sc_sum, sc_sq,
              r1(sc_gamma), r1(sc_beta))
    return jnp.transpose(out, (0, 2, 1))[:, :, :, None]


# speedup vs baseline: 1.0321x; 1.0321x over previous
import functools

import jax
import jax.numpy as jnp
from jax import lax
from jax.experimental import pallas as pl
from jax.experimental.pallas import tpu as pltpu
from jax.experimental.pallas import tpu_sc as plsc

B = 2
N = 8192
K = 16
EPS = 1e-6
TD = 1024
TL = 256
TN = 256
ND = N // TD
NL = N // TL
NK = N // TN
TC = 128
BNK = B * N * K


def _dot(a, b):
    return lax.dot_general(a, b, (((1,), (1,)), ((), ())),
                           preferred_element_type=jnp.float32)


def _leaky(x, slope):
    return jnp.where(x >= 0, x, slope * x)


def _gn_scale(ssum, ssq, count, c, gamma, beta):
    s = jnp.sum(ssum, axis=0, keepdims=True)
    q = jnp.sum(ssq, axis=0, keepdims=True)
    gper = c // 16
    gi = lax.broadcasted_iota(jnp.int32, (c, c), 0) // gper
    gj = lax.broadcasted_iota(jnp.int32, (c, c), 1) // gper
    gmat = (gi == gj).astype(jnp.float32)
    mu = _dot(s, gmat) / count
    msq = _dot(q, gmat) / count
    inv = lax.rsqrt(msq - mu * mu + EPS)
    scale = inv * gamma
    shift = beta - mu * scale
    return scale, shift


def _acc(ref, val, is_first):
    @pl.when(is_first)
    def _():
        ref[...] = val[None]

    @pl.when(jnp.logical_not(is_first))
    def _():
        ref[...] += val[None]


def _k0_body(f_ref, c_ref, w1_ref, b1_ref, scw_ref, scb_ref,
             tab_ref, ssum_ref, ssq_ref):
    t = pl.program_id(1)
    f = f_ref[0]
    c = c_ref[0]
    x1 = _leaky(_dot(f, w1_ref[...]) + b1_ref[...], 0.2)
    tab_ref[...] = jnp.concatenate(
        [x1, c, jnp.zeros((TD, TC - 35), jnp.float32)], axis=1)
    pre = _dot(f, scw_ref[...]) + scb_ref[...]
    ps = jnp.sum(pre.reshape(TD // 8, 8, 128), axis=0)
    pq = jnp.sum((pre * pre).reshape(TD // 8, 8, 128), axis=0)
    _acc(ssum_ref, ps, t == 0)
    _acc(ssq_ref, pq, t == 0)


def _k0(feats, coords, w1, b1, scw, scb):
    return pl.pallas_call(
        _k0_body,
        grid=(B, ND),
        in_specs=[
            pl.BlockSpec((1, TD, 32), lambda b, t: (b, t, 0)),
            pl.BlockSpec((1, TD, 3), lambda b, t: (b, t, 0)),
            pl.BlockSpec((32, 32), lambda b, t: (0, 0)),
            pl.BlockSpec((1, 32), lambda b, t: (0, 0)),
            pl.BlockSpec((128, 32), lambda b, t: (0, 0)),
            pl.BlockSpec((1, 128), lambda b, t: (0, 0)),
        ],
        out_specs=[
            pl.BlockSpec((TD, TC), lambda b, t: (b * ND + t, 0)),
            pl.BlockSpec((1, 8, 128), lambda b, t: (b, 0, 0)),
            pl.BlockSpec((1, 8, 128), lambda b, t: (b, 0, 0)),
        ],
        out_shape=[
            jax.ShapeDtypeStruct((B * N, TC), jnp.float32),
            jax.ShapeDtypeStruct((B, 8, 128), jnp.float32),
            jax.ShapeDtypeStruct((B, 8, 128), jnp.float32),
        ],
    )(feats, coords, w1, b1, scw, scb)


def _knn_body(r_ref, a_ref, idx_ref):
    b = pl.program_id(0)
    r = r_ref[0]
    allc = a_ref[0]
    sr = jnp.sum(r * r, axis=1)
    sa = jnp.sum(allc * allc, axis=1)
    cross = _dot(r, allc)
    d2 = sr[:, None] + sa[None, :] - 2.0 * cross
    colid = lax.broadcasted_iota(jnp.int32, (TN, N), 1)
    for k in range(K):
        a = jnp.argmin(d2, axis=1).astype(jnp.int32)
        idx_ref[0, k, :] = a + b * N
        d2 = jnp.where(colid == a[:, None], jnp.float32(jnp.inf), d2)


def _knn(coords):
    return pl.pallas_call(
        _knn_body,
        grid=(B, NK),
        in_specs=[
            pl.BlockSpec((1, TN, 3), lambda b, t: (b, t, 0)),
            pl.BlockSpec((1, N, 3), lambda b, t: (b, 0, 0)),
        ],
        out_specs=pl.BlockSpec((1, K, TN), lambda b, t: (b, 0, t)),
        out_shape=jax.ShapeDtypeStruct((B, K, N), jnp.int32),
    )(coords, coords)


_SC_NW = 32
_SC_RW = BNK // _SC_NW
_SC_SUP = 512
_SC_NSUP = _SC_RW // _SC_SUP


def _sc_gather_body(tab_ref, idx_ref, out_ref, idx_v, rows_v, sem):
    wid = lax.axis_index("s") * 2 + lax.axis_index("c")
    for s in range(_SC_NSUP):
        base = pl.multiple_of(wid * _SC_RW + s * _SC_SUP, _SC_SUP)
        irow = pl.multiple_of(base // 128, _SC_SUP // 128)
        pltpu.sync_copy(idx_ref.at[pl.ds(irow, _SC_SUP // 128)], idx_v)
        copies = []
        for j in range(_SC_SUP // 128):
            copies.append(pltpu.async_copy(
                tab_ref.at[idx_v.at[j]],
                rows_v.at[pl.ds(j * 128, 128)], sem))
        for cp in copies:
            cp.wait()
        pltpu.sync_copy(rows_v, out_ref.at[pl.ds(base, _SC_SUP)])


@functools.cache
def _sc_gather_kernel():
    return functools.partial(
        pl.kernel,
        mesh=plsc.VectorSubcoreMesh(core_axis_name="c", subcore_axis_name="s"),
        out_type=jax.ShapeDtypeStruct((BNK, TC), jnp.float32),
        scratch_types=[
            pltpu.VMEM((_SC_SUP // 128, 128), jnp.int32),
            pltpu.VMEM((_SC_SUP, TC), jnp.float32),
            pltpu.SemaphoreType.DMA,
        ],
    )(_sc_gather_body)


def _run_gather(table, idx2d):
    return _sc_gather_kernel()(table, idx2d)


def _lse_pre(c, g, w, bias):
    wn = w[:, 0:3] + w[:, 6:9]
    wnb = w[:, 3:6] - w[:, 6:9]
    wd3 = jnp.broadcast_to(w[:, 9:10], (32, 3))
    a_n = _dot(c, wn)
    c_nb = g[:, 32:35]
    a_nb = _dot(c_nb, wnb)
    c_self = jnp.broadcast_to(c[:, None, :], (TL, K, 3)).reshape(TL * K, 3)
    diff = c_self - c_nb
    dterm = _dot(diff * diff, wd3)
    a_nr = jnp.broadcast_to(a_n[:, None, :], (TL, K, 32)).reshape(TL * K, 32)
    return a_nr + a_nb + dterm + bias


def _lseA_body(c_ref, g_ref, w_ref, b_ref,
               ssum_ref, ssq_ref, fmean_ref, pre_ref):
    t = pl.program_id(1)
    g = g_ref[...]
    pre = _lse_pre(c_ref[0], g, w_ref[...], b_ref[...])
    pre_ref[...] = pre
    fmean_ref[0] = jnp.mean(g[:, 0:32].reshape(TL, K, 32), axis=1)
    ps = jnp.sum(pre.reshape(TL * K // 8, 8, 32), axis=0)
    pq = jnp.sum((pre * pre).reshape(TL * K // 8, 8, 32), axis=0)
    _acc(ssum_ref, ps, t == 0)
    _acc(ssq_ref, pq, t == 0)


def _lseA(coords, gath, w, bias):
    return pl.pallas_call(
        _lseA_body,
        grid=(B, NL),
        in_specs=[
            pl.BlockSpec((1, TL, 3), lambda b, t: (b, t, 0)),
            pl.BlockSpec((TL * K, TC), lambda b, t: (b * NL + t, 0)),
            pl.BlockSpec((32, 10), lambda b, t: (0, 0)),
            pl.BlockSpec((1, 32), lambda b, t: (0, 0)),
        ],
        out_specs=[
            pl.BlockSpec((1, 8, 32), lambda b, t: (b, 0, 0)),
            pl.BlockSpec((1, 8, 32), lambda b, t: (b, 0, 0)),
            pl.BlockSpec((1, TL, 32), lambda b, t: (b, t, 0)),
            pl.BlockSpec((TL * K, 32), lambda b, t: (b * NL + t, 0)),
        ],
        out_shape=[
            jax.ShapeDtypeStruct((B, 8, 32), jnp.float32),
            jax.ShapeDtypeStruct((B, 8, 32), jnp.float32),
            jax.ShapeDtypeStruct((B, N, 32), jnp.float32),
            jax.ShapeDtypeStruct((BNK, 32), jnp.float32),
        ],
    )(coords, gath, w, bias)


def _lseB_body(pout, pre_ref, gam_ref, bet_ref,
               ssum_ref, ssq_ref, fmean_ref, pw_ref, pb_ref,
               prep_ref, psum_ref, psq_ref):
    t = pl.program_id(1)
    scale, shift = _gn_scale(ssum_ref[0], ssq_ref[0], 2.0 * N * K, 32,
                             gam_ref[...], bet_ref[...])
    h = jax.nn.relu(pre_ref[...] * scale + shift)
    hmean = jnp.mean(h.reshape(TL, K, 32), axis=1)
    pin = jnp.concatenate([hmean, fmean_ref[0]], axis=1)
    prep = _dot(pin, pw_ref[...]) + pb_ref[...]
    prep_ref[0] = prep
    ps = jnp.sum(prep.reshape(TL // 8, 8, pout), axis=0)
    pq = jnp.sum((prep * prep).reshape(TL // 8, 8, pout), axis=0)
    _acc(psum_ref, ps, t == 0)
    _acc(psq_ref, pq, t == 0)


def _lseB(pout, pre, gam, bet, ssum, ssq, fmean, pw, pb):
    return pl.pallas_call(
        functools.partial(_lseB_body, pout),
        grid=(B, NL),
        in_specs=[
            pl.BlockSpec((TL * K, 32), lambda b, t: (b * NL + t, 0)),
            pl.BlockSpec((1, 32), lambda b, t: (0, 0)),
            pl.BlockSpec((1, 32), lambda b, t: (0, 0)),
            pl.BlockSpec((1, 8, 32), lambda b, t: (b, 0, 0)),
            pl.BlockSpec((1, 8, 32), lambda b, t: (b, 0, 0)),
            pl.BlockSpec((1, TL, 32), lambda b, t: (b, t, 0)),
            pl.BlockSpec((pout, 64), lambda b, t: (0, 0)),
            pl.BlockSpec((1, pout), lambda b, t: (0, 0)),
        ],
        out_specs=[
            pl.BlockSpec((1, TL, pout), lambda b, t: (b, t, 0)),
            pl.BlockSpec((1, 8, pout), lambda b, t: (b, 0, 0)),
            pl.BlockSpec((1, 8, pout), lambda b, t: (b, 0, 0)),
        ],
        out_shape=[
            jax.ShapeDtypeStruct((B, N, pout), jnp.float32),
            jax.ShapeDtypeStruct((B, 8, pout), jnp.float32),
            jax.ShapeDtypeStruct((B, 8, pout), jnp.float32),
        ],
    )(pre, gam, bet, ssum, ssq, fmean, pw, pb)


def _k5_body(c_ref, prep_ref, ssum_ref, ssq_ref, gam_ref, bet_ref, tab_ref):
    scale, shift = _gn_scale(ssum_ref[0], ssq_ref[0], 2.0 * N, 32,
                             gam_ref[...], bet_ref[...])
    x2 = jax.nn.relu(prep_ref[0] * scale + shift)
    tab_ref[...] = jnp.concatenate(
        [x2, c_ref[0], jnp.zeros((TD, TC - 35), jnp.float32)], axis=1)


def _k5(coords, prep, ssum, ssq, gam, bet):
    return pl.pallas_call(
        _k5_body,
        grid=(B, ND),
        in_specs=[
            pl.BlockSpec((1, TD, 3), lambda b, t: (b, t, 0)),
            pl.BlockSpec((1, TD, 32), lambda b, t: (b, t, 0)),
            pl.BlockSpec((1, 8, 32), lambda b, t: (b, 0, 0)),
            pl.BlockSpec((1, 8, 32), lambda b, t: (b, 0, 0)),
            pl.BlockSpec((1, 32), lambda b, t: (0, 0)),
            pl.BlockSpec((1, 32), lambda b, t: (0, 0)),
        ],
        out_specs=pl.BlockSpec((TD, TC), lambda b, t: (b * ND + t, 0)),
        out_shape=jax.ShapeDtypeStruct((B * N, TC), jnp.float32),
    )(coords, prep, ssum, ssq, gam, bet)


def _k9_body(prep_ref, p2s_ref, p2q_ref, g2_ref, b2_ref,
             m2w_ref, m2b_ref, f_ref, scw_ref, scb_ref,
             scs_ref, scq_ref, scg_ref, scbt_ref, out_ref):
    scale, shift = _gn_scale(p2s_ref[0], p2q_ref[0], 4.0 * N, 64,
                             g2_ref[...], b2_ref[...])
    x3 = jax.nn.relu(prep_ref[0] * scale + shift)
    main = _dot(x3, m2w_ref[...]) + m2b_ref[...]
    pre_sc = _dot(f_ref[0], scw_ref[...]) + scb_ref[...]
    scale2, shift2 = _gn_scale(scs_ref[0], scq_ref[0], 8.0 * N, 128,
                               scg_ref[...], scbt_ref[...])
    scn = pre_sc * scale2 + shift2
    out_ref[0] = _leaky(main + scn, 0.01)


def _k9(prep2, p2s, p2q, g2, b2, m2w, m2b, feats, scw, scb, scs, scq,
        scg, scbt):
    return pl.pallas_call(
        _k9_body,
        grid=(B, ND),
        in_specs=[
            pl.BlockSpec((1, TD, 64), lambda b, t: (b, t, 0)),
            pl.BlockSpec((1, 8, 64), lambda b, t: (b, 0, 0)),
            pl.BlockSpec((1, 8, 64), lambda b, t: (b, 0, 0)),
            pl.BlockSpec((1, 64), lambda b, t: (0, 0)),
            pl.BlockSpec((1, 64), lambda b, t: (0, 0)),
            pl.BlockSpec((128, 64), lambda b, t: (0, 0)),
            pl.BlockSpec((1, 128), lambda b, t: (0, 0)),
            pl.BlockSpec((1, TD, 32), lambda b, t: (b, t, 0)),
            pl.BlockSpec((128, 32), lambda b, t: (0, 0)),
            pl.BlockSpec((1, 128), lambda b, t: (0, 0)),
            pl.BlockSpec((1, 8, 128), lambda b, t: (b, 0, 0)),
            pl.BlockSpec((1, 8, 128), lambda b, t: (b, 0, 0)),
            pl.BlockSpec((1, 128), lambda b, t: (0, 0)),
            pl.BlockSpec((1, 128), lambda b, t: (0, 0)),
        ],
        out_specs=pl.BlockSpec((1, TD, 128), lambda b, t: (b, t, 0)),
        out_shape=jax.ShapeDtypeStruct((B, N, 128), jnp.float32),
    )(prep2, p2s, p2q, g2, b2, m2w, m2b, feats, scw, scb, scs, scq,
      scg, scbt)


def kernel(coords, features, mlp1_W, mlp1_b, lse1_W, lse1_b, lse1_gamma,
           lse1_beta, pool1_W, pool1_b, pool1_gamma, pool1_beta, lse2_W,
           lse2_b, lse2_gamma, lse2_beta, pool2_W, pool2_b, pool2_gamma,
           pool2_beta, mlp2_W, mlp2_b, sc_W, sc_b, sc_gamma, sc_beta):
    r1 = lambda v: v.reshape(1, -1)
    feats = jnp.transpose(features[:, :, :, 0], (0, 2, 1))

    table1, sc_sum, sc_sq = _k0(feats, coords, mlp1_W, r1(mlp1_b),
                                sc_W, r1(sc_b))
    idx_kn = _knn(coords)
    idx2d = jnp.transpose(idx_kn, (0, 2, 1)).reshape(BNK // 128, 128)

    gath1 = _run_gather(table1, idx2d)
    s1, q1, fm1, pre1 = _lseA(coords, gath1, lse1_W, r1(lse1_b))
    prep1, p1s, p1q = _lseB(32, pre1,
                            r1(lse1_gamma), r1(lse1_beta), s1, q1, fm1,
                            pool1_W, r1(pool1_b))
    table2 = _k5(coords, prep1, p1s, p1q, r1(pool1_gamma), r1(pool1_beta))

    gath2 = _run_gather(table2, idx2d)
    s2, q2, fm2, pre2 = _lseA(coords, gath2, lse2_W, r1(lse2_b))
    prep2, p2s, p2q = _lseB(64, pre2,
                            r1(lse2_gamma), r1(lse2_beta), s2, q2, fm2,
                            pool2_W, r1(pool2_b))

    out = _k9(prep2, p2s, p2q, r1(pool2_gamma), r1(pool2_beta), mlp2_W,
              r1(mlp2_b), feats, sc_W, r1(sc_b), sc_sum, sc_sq,
              r1(sc_gamma), r1(sc_beta))
    return jnp.transpose(out, (0, 2, 1))[:, :, :, None]
